# Initial kernel scaffold; baseline (speedup 1.0000x reference)
import jax
import jax.numpy as jnp
from jax.experimental import pallas as pl


def kernel(x, condition, quant_emb, cond_emb_weight):
    raise NotImplementedError("write your pallas kernel here")


# speedup vs baseline: 5.6489x; 5.6489x over previous
import functools

import jax
import jax.numpy as jnp
from jax import lax
from jax.experimental import pallas as pl
from jax.experimental.pallas import tpu as pltpu
from jax.experimental.pallas import tpu_sc as plsc

B, C, T = 16, 8, 2048
QUANT_LEVELS, QUANT_EMB = 1024, 64
NUM_CLASSES, CLASS_EMB = 1000, 128

NW = 32
ROWS_OUT = B * T * C
ROWS_COND = B * T
OUT_PER_W = ROWS_OUT // NW
COND_PER_W = ROWS_COND // NW
OUT_CHUNK = 1024
L = 16

_mesh = plsc.VectorSubcoreMesh(core_axis_name="c", subcore_axis_name="s")


@functools.partial(
    pl.kernel,
    mesh=_mesh,
    out_type=(
        jax.ShapeDtypeStruct((ROWS_OUT, QUANT_EMB), jnp.float32),
        jax.ShapeDtypeStruct((ROWS_COND, CLASS_EMB), jnp.float32),
    ),
    scratch_types=[
        pltpu.VMEM((OUT_CHUNK // 128, 128), jnp.int32),
        pltpu.VMEM((OUT_CHUNK, QUANT_EMB), jnp.float32),
        pltpu.VMEM((COND_PER_W // 128, 128), jnp.int32),
        pltpu.VMEM((128, CLASS_EMB), jnp.float32),
        pltpu.SemaphoreType.DMA,
    ],
    compiler_params=pltpu.CompilerParams(use_tc_tiling_on_sc=False),
)
def _sc_gather(xp_hbm, cidx_hbm, qtab_hbm, wtab_hbm, out_hbm, crows_hbm,
               idx_v, rows_v, cidx_v, crows_v, sem):
    wid = lax.axis_index("s") * 2 + lax.axis_index("c")
    pattern = (lax.iota(jnp.int32, L) % C) * QUANT_LEVELS

    def out_step(j, _):
        base8 = wid * (OUT_PER_W // 128) + j * (OUT_CHUNK // 128)
        pltpu.sync_copy(xp_hbm.at[pl.ds(base8, OUT_CHUNK // 128)], idx_v)
        for r in range(OUT_CHUNK // 128):
            for g in range(128 // L):
                sl = pl.ds(g * L, L)
                idx_v[r, sl] = idx_v[r, sl] + pattern
        cps = [
            pltpu.async_copy(qtab_hbm.at[idx_v.at[r]],
                             rows_v.at[pl.ds(r * 128, 128)], sem)
            for r in range(OUT_CHUNK // 128)
        ]
        for cp in cps:
            cp.wait()
        pltpu.sync_copy(rows_v, out_hbm.at[pl.ds(base8 * 128, OUT_CHUNK)])
        return 0

    lax.fori_loop(0, OUT_PER_W // OUT_CHUNK, out_step, 0)

    pltpu.sync_copy(cidx_hbm.at[pl.ds(wid * (COND_PER_W // 128),
                                      COND_PER_W // 128)], cidx_v)
    for r in range(COND_PER_W // 128):
        pltpu.async_copy(wtab_hbm.at[cidx_v.at[r]], crows_v, sem).wait()
        pltpu.sync_copy(crows_v,
                        crows_hbm.at[pl.ds(wid * COND_PER_W + r * 128, 128)])


TB = 256


def _tc_expand_body(crows_ref, cond_ref, out_ref, scratch_ref):
    c = pl.program_id(2)

    @pl.when(c == 0)
    def _():
        rows = crows_ref[0]
        mask = (cond_ref[0] > 0).astype(jnp.float32)
        scratch_ref[...] = rows.T * mask

    out_ref[0] = scratch_ref[...]


def _tc_expand(crows3, condition):
    return pl.pallas_call(
        _tc_expand_body,
        grid=(B, T // TB, C),
        in_specs=[
            pl.BlockSpec((1, TB, CLASS_EMB), lambda b, t, c: (b, t, 0)),
            pl.BlockSpec((1, 1, TB), lambda b, t, c: (b, 0, t)),
        ],
        out_specs=pl.BlockSpec((1, CLASS_EMB, TB),
                               lambda b, t, c: (b * C + c, 0, t)),
        out_shape=jax.ShapeDtypeStruct((B * C, CLASS_EMB, T), jnp.float32),
        scratch_shapes=[pltpu.VMEM((CLASS_EMB, TB), jnp.float32)],
    )(crows3, condition)


def kernel(x, condition, quant_emb, cond_emb_weight):
    xp = jnp.transpose(x, (0, 2, 1)).reshape(ROWS_OUT // 128, 128)
    cidx = condition.reshape(ROWS_COND // 128, 128)
    qtab = quant_emb.reshape(C * QUANT_LEVELS, QUANT_EMB)

    out_flat, crows = _sc_gather(xp, cidx, qtab, cond_emb_weight)
    cond = _tc_expand(crows.reshape(B, T, CLASS_EMB), condition)
    return out_flat.reshape(B * T, C, QUANT_EMB), cond


# speedup vs baseline: 11.0259x; 1.9519x over previous
import functools

import jax
import jax.numpy as jnp
from jax import lax
from jax.experimental import pallas as pl
from jax.experimental.pallas import tpu as pltpu
from jax.experimental.pallas import tpu_sc as plsc

B, C, T = 16, 8, 2048
QUANT_LEVELS, QUANT_EMB = 1024, 64
NUM_CLASSES, CLASS_EMB = 1000, 128

NW = 32
ROWS_OUT = B * T * C
ROWS_COND = B * T
OUT_PER_W = ROWS_OUT // NW
COND_PER_W = ROWS_COND // NW
OUT_CHUNK = 1024
L = 16

_mesh = plsc.VectorSubcoreMesh(core_axis_name="c", subcore_axis_name="s")


@functools.partial(
    pl.kernel,
    mesh=_mesh,
    out_type=(
        jax.ShapeDtypeStruct((ROWS_OUT, QUANT_EMB), jnp.float32),
        jax.ShapeDtypeStruct((ROWS_COND, CLASS_EMB), jnp.float32),
    ),
    scratch_types=[
        pltpu.VMEM((OUT_CHUNK // 128, 128), jnp.int32),
        pltpu.VMEM((OUT_CHUNK, QUANT_EMB), jnp.float32),
        pltpu.VMEM((COND_PER_W // 128, 128), jnp.int32),
        pltpu.VMEM((128, CLASS_EMB), jnp.float32),
        pltpu.SemaphoreType.DMA,
    ],
    compiler_params=pltpu.CompilerParams(use_tc_tiling_on_sc=False),
)
def _sc_gather(xp_hbm, cidx_hbm, qtab_hbm, wtab_hbm, out_hbm, crows_hbm,
               idx_v, rows_v, cidx_v, crows_v, sem):
    wid = lax.axis_index("s") * 2 + lax.axis_index("c")
    pattern = (lax.iota(jnp.int32, L) % C) * QUANT_LEVELS

    def out_step(j, _):
        base8 = wid * (OUT_PER_W // 128) + j * (OUT_CHUNK // 128)
        pltpu.sync_copy(xp_hbm.at[pl.ds(base8, OUT_CHUNK // 128)], idx_v)
        for r in range(OUT_CHUNK // 128):
            for g in range(128 // L):
                sl = pl.ds(g * L, L)
                idx_v[r, sl] = idx_v[r, sl] + pattern
        cps = [
            pltpu.async_copy(qtab_hbm.at[idx_v.at[r]],
                             rows_v.at[pl.ds(r * 128, 128)], sem)
            for r in range(OUT_CHUNK // 128)
        ]
        for cp in cps:
            cp.wait()
        pltpu.sync_copy(rows_v, out_hbm.at[pl.ds(base8 * 128, OUT_CHUNK)])
        return 0

    lax.fori_loop(0, OUT_PER_W // OUT_CHUNK, out_step, 0)

    pltpu.sync_copy(cidx_hbm.at[pl.ds(wid * (COND_PER_W // 128),
                                      COND_PER_W // 128)], cidx_v)
    for r in range(COND_PER_W // 128):
        pltpu.async_copy(wtab_hbm.at[cidx_v.at[r]], crows_v, sem).wait()
        pltpu.sync_copy(crows_v,
                        crows_hbm.at[pl.ds(wid * COND_PER_W + r * 128, 128)])


TB = 512


def _tc_expand_body(crows_ref, cond_ref, out_ref):
    rows = crows_ref[0]
    mask = (cond_ref[0] > 0).astype(jnp.float32)
    val = rows.T * mask
    for c in range(C):
        out_ref[c] = val


def _tc_expand(crows3, condition):
    return pl.pallas_call(
        _tc_expand_body,
        grid=(B, T // TB),
        in_specs=[
            pl.BlockSpec((1, TB, CLASS_EMB), lambda b, t: (b, t, 0)),
            pl.BlockSpec((1, 1, TB), lambda b, t: (b, 0, t)),
        ],
        out_specs=pl.BlockSpec((C, CLASS_EMB, TB), lambda b, t: (b, 0, t)),
        out_shape=jax.ShapeDtypeStruct((B * C, CLASS_EMB, T), jnp.float32),
    )(crows3, condition)


def kernel(x, condition, quant_emb, cond_emb_weight):
    xp = jnp.transpose(x, (0, 2, 1)).reshape(ROWS_OUT // 128, 128)
    cidx = condition.reshape(ROWS_COND // 128, 128)
    qtab = quant_emb.reshape(C * QUANT_LEVELS, QUANT_EMB)

    out_flat, crows = _sc_gather(xp, cidx, qtab, cond_emb_weight)
    cond = _tc_expand(crows.reshape(B, T, CLASS_EMB), condition)
    return out_flat.reshape(B * T, C, QUANT_EMB), cond
